# baseline (speedup 1.0000x reference)
import functools

import jax
import jax.numpy as jnp
from jax.experimental import pallas as pl
from jax.experimental.pallas import tpu as pltpu

_TOPK = 8
_N_GROUPS = 8
_TOPK_GROUPS = 4
_GROUP_CRIT_K = 2
_ROUTE_SCALE = 2.5
_N_EXPERTS = 64
_DIM = 768
_TOKENS = 32768

_BT = 256
_NEG = float("-inf")


def _moe_gate_block(x_ref, w_ref, wout_ref, iout_ref):
    x = x_ref[...]
    w = w_ref[...]
    scores = jax.lax.dot_general(
        x, w, (((1,), (1,)), ((), ())), preferred_element_type=jnp.float32
    )
    scores = jax.nn.sigmoid(scores)

    epg = _N_EXPERTS // _N_GROUPS
    i_g = jax.lax.broadcasted_iota(jnp.int32, (_BT, epg), 1)

    gs_list = []
    for g in range(_N_GROUPS):
        sg = scores[:, g * epg:(g + 1) * epg]
        m1 = jnp.max(sg, axis=-1, keepdims=True)
        am = jnp.min(jnp.where(sg == m1, i_g, epg), axis=-1, keepdims=True)
        m2 = jnp.max(jnp.where(i_g == am, _NEG, sg), axis=-1, keepdims=True)
        gs_list.append(m1 + m2)
    gs = jnp.concatenate(gs_list, axis=-1)

    sel = jnp.zeros((_BT, _N_GROUPS), dtype=jnp.bool_)
    for _ in range(_TOPK_GROUPS):
        m = jnp.max(gs, axis=-1, keepdims=True)
        am = jnp.min(jnp.where(gs == m, i_g, _N_GROUPS), axis=-1, keepdims=True)
        hit = i_g == am
        sel = jnp.logical_or(sel, hit)
        gs = jnp.where(hit, _NEG, gs)

    masked = jnp.concatenate(
        [
            jnp.where(sel[:, g:g + 1], scores[:, g * epg:(g + 1) * epg], _NEG)
            for g in range(_N_GROUPS)
        ],
        axis=-1,
    )

    i_e = jax.lax.broadcasted_iota(jnp.int32, (_BT, _N_EXPERTS), 1)
    idx_list, val_list = [], []
    for _ in range(_TOPK):
        m = jnp.max(masked, axis=-1, keepdims=True)
        am = jnp.min(jnp.where(masked == m, i_e, _N_EXPERTS), axis=-1, keepdims=True)
        idx_list.append(am)
        val_list.append(m)
        masked = jnp.where(i_e == am, _NEG, masked)
    idx = jnp.concatenate(idx_list, axis=-1)
    vals = jnp.concatenate(val_list, axis=-1)

    wts = vals * (_ROUTE_SCALE / jnp.sum(vals, axis=-1, keepdims=True))
    wout_ref[...] = wts
    iout_ref[...] = idx


@jax.jit
def kernel(x, weight):
    grid = (_TOKENS // _BT,)
    wout, iout = pl.pallas_call(
        _moe_gate_block,
        grid=grid,
        in_specs=[
            pl.BlockSpec((_BT, _DIM), lambda i: (i, 0)),
            pl.BlockSpec((_N_EXPERTS, _DIM), lambda i: (0, 0)),
        ],
        out_specs=[
            pl.BlockSpec((_BT, _TOPK), lambda i: (i, 0)),
            pl.BlockSpec((_BT, _TOPK), lambda i: (i, 0)),
        ],
        out_shape=[
            jax.ShapeDtypeStruct((_TOKENS, _TOPK), jnp.float32),
            jax.ShapeDtypeStruct((_TOKENS, _TOPK), jnp.int32),
        ],
        compiler_params=pltpu.CompilerParams(
            dimension_semantics=("arbitrary",),
        ),
    )(x, weight)
    return wout, iout


# speedup vs baseline: 7.2918x; 7.2918x over previous
import jax
import jax.numpy as jnp
import numpy as np
from jax.experimental import pallas as pl
from jax.experimental.pallas import tpu as pltpu

_TOPK = 8
_N_GROUPS = 8
_TOPK_GROUPS = 4
_ROUTE_SCALE = 2.5
_N_EXPERTS = 64
_DIM = 768
_TOKENS = 32768

_BT = 256
_SENT_BITS = int(np.float32(-1e30).view(np.int32))


def _moe_gate_block(x_ref, w_ref, wout_ref, iout_ref):
    x = x_ref[...]
    w = w_ref[...]
    st = jax.lax.dot_general(
        w, x, (((1,), (1,)), ((), ())), preferred_element_type=jnp.float32
    )
    st = jax.nn.sigmoid(st)

    row = jax.lax.broadcasted_iota(jnp.int32, (_N_EXPERTS, _BT), 0)

    m1 = st
    m2 = None
    for k in (1, 2, 4):
        bit = (row & k) == 0
        pm1 = jnp.where(bit, pltpu.roll(m1, _N_EXPERTS - k, 0), pltpu.roll(m1, k, 0))
        if m2 is None:
            m2 = jnp.minimum(m1, pm1)
        else:
            pm2 = jnp.where(bit, pltpu.roll(m2, _N_EXPERTS - k, 0), pltpu.roll(m2, k, 0))
            m2 = jnp.maximum(jnp.minimum(m1, pm1), jnp.maximum(m2, pm2))
        m1 = jnp.maximum(m1, pm1)
    gs = m1 + m2

    g = row >> 3
    rank = jnp.zeros((_N_EXPERTS, _BT), dtype=jnp.int32)
    for j in range(1, _N_GROUPS):
        other = pltpu.roll(gs, _N_EXPERTS - 8 * j, 0)
        og_lt = ((g + j) & 7) < g
        beats = (other > gs) | ((other == gs) & og_lt)
        rank = rank + jnp.where(beats, 1, 0)
    sel = rank < _TOPK_GROUPS

    kb = st.view(jnp.int32)
    keys = ((kb & ~63) | (63 - row)).view(jnp.float32)
    sent = (_SENT_BITS | (63 - row)).view(jnp.float32)
    masked = jnp.where(sel, keys, sent)

    picked = []
    for _ in range(_TOPK):
        m = jnp.max(masked, axis=0, keepdims=True)
        picked.append(m)
        masked = jnp.where(masked == m, sent, masked)

    mk = jnp.concatenate(picked, axis=0)
    idx = 63 - (mk.view(jnp.int32) & 63)
    wts = mk * (_ROUTE_SCALE / jnp.sum(mk, axis=0, keepdims=True))

    wout_ref[...] = wts.T
    iout_ref[...] = idx.T


@jax.jit
def kernel(x, weight):
    grid = (_TOKENS // _BT,)
    wout, iout = pl.pallas_call(
        _moe_gate_block,
        grid=grid,
        in_specs=[
            pl.BlockSpec((_BT, _DIM), lambda i: (i, 0)),
            pl.BlockSpec((_N_EXPERTS, _DIM), lambda i: (0, 0)),
        ],
        out_specs=[
            pl.BlockSpec((_BT, _TOPK), lambda i: (i, 0)),
            pl.BlockSpec((_BT, _TOPK), lambda i: (i, 0)),
        ],
        out_shape=[
            jax.ShapeDtypeStruct((_TOKENS, _TOPK), jnp.float32),
            jax.ShapeDtypeStruct((_TOKENS, _TOPK), jnp.int32),
        ],
        compiler_params=pltpu.CompilerParams(
            dimension_semantics=("arbitrary",),
        ),
    )(x, weight)
    return wout, iout
